# baseline (speedup 1.0000x reference)
import functools

import jax
import jax.numpy as jnp
from jax import lax
from jax.experimental import pallas as pl
from jax.experimental.pallas import tpu as pltpu
from jax.experimental.pallas import tpu_sc as plsc

N = 10000
E = 320000
D = 128
EPS = 1e-5

NC = 2
NS = 16
NW = NC * NS
EPT = E // NW
K = 40
NCHUNK = EPT // K
NB = 3
M = (NCHUNK - 4) // NB
RPT = 624
TAIL = N - NS * RPT


def _lin_body(x_ref, w_ref, o_ref):
    o_ref[...] = lax.dot_general(
        x_ref[...], w_ref[...], (((1,), (1,)), ((), ())),
        preferred_element_type=jnp.float32)


def _bn(y, g, b):
    m = jnp.mean(y, axis=0, keepdims=True)
    c = y - m
    v = jnp.mean(c * c, axis=0, keepdims=True)
    return c * lax.rsqrt(v + EPS) * g + b


def _post_body(p_ref, x_ref, b_ref, bng_ref, bnb_ref, n1g_ref, n1b_ref,
               w1_ref, b1_ref, w2_ref, b2_ref, n2g_ref, n2b_ref, o_ref):
    agg = p_ref[0] + p_ref[1] + b_ref[...]
    out = jnp.maximum(_bn(agg, bng_ref[...], bnb_ref[...]), 0.0) + x_ref[...]
    out = _bn(out, n1g_ref[...], n1b_ref[...])
    ff = jnp.maximum(
        lax.dot_general(out, w1_ref[...], (((1,), (1,)), ((), ())),
                        preferred_element_type=jnp.float32) + b1_ref[...], 0.0)
    ff = lax.dot_general(ff, w2_ref[...], (((1,), (1,)), ((), ())),
                         preferred_element_type=jnp.float32) + b2_ref[...]
    o_ref[...] = _bn(out + ff, n2g_ref[...], n2b_ref[...])


def _sc_body(h_hbm, src_hbm, dst_hbm, ea_hbm, out_hbm,
             src_v, dst0, dst1, dst2, hrow0, hrow1, hrow2, ea0, ea1, ea2,
             agg_sh, sem0, sem1, sem2, ssem0, ssem1, ssem2):
    cid = lax.axis_index("c")
    sid = lax.axis_index("s")
    wid = cid * NS + sid

    zf = jnp.zeros((16,), jnp.float32)

    @plsc.parallel_loop(0, K, step=1, unroll=4)
    def _zrow(r):
        for l in range(D // 16):
            ea0[r, pl.ds(l * 16, 16)] = zf

    for t in range(RPT // K):
        pltpu.sync_copy(ea0, agg_sh.at[pl.ds(sid * RPT + t * K, K)])
    pltpu.sync_copy(ea0.at[pl.ds(0, RPT % K)],
                    agg_sh.at[pl.ds(sid * RPT + (RPT // K) * K, RPT % K)])

    @pl.when(sid == NS - 1)
    def _():
        pltpu.sync_copy(ea0.at[pl.ds(0, TAIL)],
                        agg_sh.at[pl.ds(NS * RPT, TAIL)])

    plsc.subcore_barrier()

    pltpu.sync_copy(src_hbm.at[wid], src_v)

    bufs = ((dst0, hrow0, ea0, sem0, ssem0),
            (dst1, hrow1, ea1, sem1, ssem1),
            (dst2, hrow2, ea2, sem2, ssem2))

    def start(i, b, drain=True):
        dst_b, hrow_b, ea_b, sem_b, ssem_b = bufs[b]
        if drain:
            pltpu.make_async_copy(ea_b, agg_sh.at[dst_b], ssem_b).wait()
        eo = wid * EPT + i * K
        pltpu.async_copy(h_hbm.at[src_v.at[pl.ds(i * K, K)]], hrow_b, sem_b)
        pltpu.async_copy(ea_hbm.at[pl.ds(eo, K)], ea_b, sem_b)
        pltpu.async_copy(dst_hbm.at[pl.ds(eo, K)], dst_b, sem_b)

    def proc(b):
        dst_b, hrow_b, ea_b, sem_b, ssem_b = bufs[b]
        pltpu.make_async_copy(h_hbm.at[src_v.at[pl.ds(0, K)]], hrow_b,
                              sem_b).wait()
        pltpu.make_async_copy(ea_hbm.at[pl.ds(0, K)], ea_b, sem_b).wait()
        pltpu.make_async_copy(dst_hbm.at[pl.ds(0, K)], dst_b, sem_b).wait()

        @plsc.parallel_loop(0, K, step=1, unroll=8)
        def _row(rr):
            for g in range(D // 16):
                sl = pl.ds(g * 16, 16)
                ea_b[rr, sl] = jnp.maximum(hrow_b[rr, sl] + ea_b[rr, sl], 0.0)

        pltpu.async_copy(ea_b, agg_sh.at[dst_b], ssem_b, add=True)

    def drain(b):
        dst_b, hrow_b, ea_b, sem_b, ssem_b = bufs[b]
        pltpu.make_async_copy(ea_b, agg_sh.at[dst_b], ssem_b).wait()

    start(0, 0, drain=False)
    start(1, 1, drain=False)

    def body(g, carry):
        i = NB * g
        proc(0)
        start(i + 2, 2, drain=False)
        proc(1)
        start(i + 3, 0)
        proc(2)
        start(i + 4, 1)
        return carry

    body(0, 0)

    def body_drain(g, carry):
        i = NB * g
        proc(0)
        start(i + 2, 2)
        proc(1)
        start(i + 3, 0)
        proc(2)
        start(i + 4, 1)
        return carry

    lax.fori_loop(1, M, body_drain, 0)
    i = NB * M
    proc(0)
    start(i + 2, 2)
    proc(1)
    start(i + 3, 0)
    proc(2)
    proc(0)
    drain(1)
    drain(2)
    drain(0)
    plsc.subcore_barrier()

    r0 = sid * RPT
    pltpu.sync_copy(agg_sh.at[pl.ds(r0, RPT)], out_hbm.at[cid, pl.ds(r0, RPT)])

    @pl.when(sid == NS - 1)
    def _():
        pltpu.sync_copy(agg_sh.at[pl.ds(NS * RPT, TAIL)],
                        out_hbm.at[cid, pl.ds(NS * RPT, TAIL)])


_sc_propagate = functools.partial(
    pl.kernel,
    out_type=jax.ShapeDtypeStruct((NC, N, D), jnp.float32),
    mesh=plsc.VectorSubcoreMesh(core_axis_name="c", subcore_axis_name="s"),
    scratch_types=[
        pltpu.VMEM((EPT,), jnp.int32),
        pltpu.VMEM((K,), jnp.int32),
        pltpu.VMEM((K,), jnp.int32),
        pltpu.VMEM((K,), jnp.int32),
        pltpu.VMEM((K, D), jnp.float32),
        pltpu.VMEM((K, D), jnp.float32),
        pltpu.VMEM((K, D), jnp.float32),
        pltpu.VMEM((K, D), jnp.float32),
        pltpu.VMEM((K, D), jnp.float32),
        pltpu.VMEM((K, D), jnp.float32),
        pltpu.VMEM_SHARED((N, D), jnp.float32),
        pltpu.SemaphoreType.DMA,
        pltpu.SemaphoreType.DMA,
        pltpu.SemaphoreType.DMA,
        pltpu.SemaphoreType.DMA,
        pltpu.SemaphoreType.DMA,
        pltpu.SemaphoreType.DMA,
    ],
)(_sc_body)


def kernel(x, edge_index, edge_attr, W, b, bn_g, bn_b, n1_g, n1_b, W1, b1,
           W2, b2, n2_g, n2_b):
    src = edge_index[0].astype(jnp.int32).reshape(NW, EPT)
    dst = edge_index[1].astype(jnp.int32)

    h = pl.pallas_call(
        _lin_body,
        out_shape=jax.ShapeDtypeStruct((N, D), jnp.float32),
    )(x, W)

    partials = _sc_propagate(h, src, dst, edge_attr)

    out = pl.pallas_call(
        _post_body,
        out_shape=jax.ShapeDtypeStruct((N, D), jnp.float32),
    )(partials, x, b.reshape(1, D), bn_g.reshape(1, D), bn_b.reshape(1, D),
      n1_g.reshape(1, D), n1_b.reshape(1, D), W1, b1.reshape(1, 2 * D),
      W2, b2.reshape(1, D), n2_g.reshape(1, D), n2_b.reshape(1, D))
    return out


# speedup vs baseline: 1.0267x; 1.0267x over previous
import functools

import jax
import jax.numpy as jnp
from jax import lax
from jax.experimental import pallas as pl
from jax.experimental.pallas import tpu as pltpu
from jax.experimental.pallas import tpu_sc as plsc

N = 10000
E = 320000
D = 128
EPS = 1e-5

NC = 2
NS = 16
NW = NC * NS
EPT = E // NW
K = 40
NCHUNK = EPT // K
NB = 3
M = (NCHUNK - 4) // NB
RPT = 624
TAIL = N - NS * RPT


def _lin_body(x_ref, w_ref, o_ref):
    o_ref[...] = lax.dot_general(
        x_ref[...], w_ref[...], (((1,), (1,)), ((), ())),
        preferred_element_type=jnp.float32)


def _bn(y, g, b):
    m = jnp.mean(y, axis=0, keepdims=True)
    c = y - m
    v = jnp.mean(c * c, axis=0, keepdims=True)
    return c * lax.rsqrt(v + EPS) * g + b


def _post_body(p_ref, x_ref, b_ref, bng_ref, bnb_ref, n1g_ref, n1b_ref,
               w1_ref, b1_ref, w2_ref, b2_ref, n2g_ref, n2b_ref, o_ref):
    agg = p_ref[0] + p_ref[1] + b_ref[...]
    out = jnp.maximum(_bn(agg, bng_ref[...], bnb_ref[...]), 0.0) + x_ref[...]
    out = _bn(out, n1g_ref[...], n1b_ref[...])
    ff = jnp.maximum(
        lax.dot_general(out, w1_ref[...], (((1,), (1,)), ((), ())),
                        preferred_element_type=jnp.float32) + b1_ref[...], 0.0)
    ff = lax.dot_general(ff, w2_ref[...], (((1,), (1,)), ((), ())),
                         preferred_element_type=jnp.float32) + b2_ref[...]
    o_ref[...] = _bn(out + ff, n2g_ref[...], n2b_ref[...])


def _sc_body(h_hbm, src_hbm, dst_hbm, ea_hbm, out_hbm,
             src_v, dst0, dst1, dst2, hrow0, hrow1, hrow2, ea0, ea1, ea2,
             agg_sh, sem0, sem1, sem2, ssem0, ssem1, ssem2):
    cid = lax.axis_index("c")
    sid = lax.axis_index("s")
    wid = cid * NS + sid

    zf = jnp.zeros((16,), jnp.float32)

    @plsc.parallel_loop(0, K, step=1, unroll=4)
    def _zrow(r):
        for l in range(D // 16):
            ea0[r, pl.ds(l * 16, 16)] = zf

    for t in range(RPT // K):
        pltpu.sync_copy(ea0, agg_sh.at[pl.ds(sid * RPT + t * K, K)])
    pltpu.sync_copy(ea0.at[pl.ds(0, RPT % K)],
                    agg_sh.at[pl.ds(sid * RPT + (RPT // K) * K, RPT % K)])

    @pl.when(sid == NS - 1)
    def _():
        pltpu.sync_copy(ea0.at[pl.ds(0, TAIL)],
                        agg_sh.at[pl.ds(NS * RPT, TAIL)])

    plsc.subcore_barrier()

    pltpu.sync_copy(src_hbm.at[wid], src_v)

    bufs = ((dst0, hrow0, ea0, sem0, ssem0),
            (dst1, hrow1, ea1, sem1, ssem1),
            (dst2, hrow2, ea2, sem2, ssem2))

    def start(i, b, drain=True):
        dst_b, hrow_b, ea_b, sem_b, ssem_b = bufs[b]
        if drain:
            pltpu.make_async_copy(ea_b, agg_sh.at[dst_b], ssem_b).wait()
        eo = wid * EPT + i * K
        pltpu.async_copy(h_hbm.at[src_v.at[pl.ds(i * K, K)]], hrow_b, sem_b)
        pltpu.async_copy(ea_hbm.at[pl.ds(eo, K)], ea_b, sem_b)
        pltpu.async_copy(dst_hbm.at[pl.ds(eo, K)], dst_b, sem_b)

    def proc(b):
        dst_b, hrow_b, ea_b, sem_b, ssem_b = bufs[b]
        pltpu.make_async_copy(h_hbm.at[src_v.at[pl.ds(0, K)]], hrow_b,
                              sem_b).wait()
        pltpu.make_async_copy(ea_hbm.at[pl.ds(0, K)], ea_b, sem_b).wait()
        pltpu.make_async_copy(dst_hbm.at[pl.ds(0, K)], dst_b, sem_b).wait()

        @plsc.parallel_loop(0, K, step=1, unroll=4)
        def _row(rr):
            for g in range(D // 16):
                sl = pl.ds(g * 16, 16)
                ea_b[rr, sl] = jnp.maximum(hrow_b[rr, sl] + ea_b[rr, sl], 0.0)

        pltpu.async_copy(ea_b, agg_sh.at[dst_b], ssem_b, add=True)

    def drain(b):
        dst_b, hrow_b, ea_b, sem_b, ssem_b = bufs[b]
        pltpu.make_async_copy(ea_b, agg_sh.at[dst_b], ssem_b).wait()

    start(0, 0, drain=False)
    start(1, 1, drain=False)

    def body(g, carry):
        i = NB * g
        proc(0)
        start(i + 2, 2, drain=False)
        proc(1)
        start(i + 3, 0)
        proc(2)
        start(i + 4, 1)
        return carry

    body(0, 0)

    def body_drain(g, carry):
        i = NB * g
        proc(0)
        start(i + 2, 2)
        proc(1)
        start(i + 3, 0)
        proc(2)
        start(i + 4, 1)
        return carry

    lax.fori_loop(1, M, body_drain, 0)
    i = NB * M
    proc(0)
    start(i + 2, 2)
    proc(1)
    start(i + 3, 0)
    proc(2)
    proc(0)
    drain(1)
    drain(2)
    drain(0)
    plsc.subcore_barrier()

    r0 = sid * RPT
    pltpu.sync_copy(agg_sh.at[pl.ds(r0, RPT)], out_hbm.at[cid, pl.ds(r0, RPT)])

    @pl.when(sid == NS - 1)
    def _():
        pltpu.sync_copy(agg_sh.at[pl.ds(NS * RPT, TAIL)],
                        out_hbm.at[cid, pl.ds(NS * RPT, TAIL)])


_sc_propagate = functools.partial(
    pl.kernel,
    out_type=jax.ShapeDtypeStruct((NC, N, D), jnp.float32),
    mesh=plsc.VectorSubcoreMesh(core_axis_name="c", subcore_axis_name="s"),
    scratch_types=[
        pltpu.VMEM((EPT,), jnp.int32),
        pltpu.VMEM((K,), jnp.int32),
        pltpu.VMEM((K,), jnp.int32),
        pltpu.VMEM((K,), jnp.int32),
        pltpu.VMEM((K, D), jnp.float32),
        pltpu.VMEM((K, D), jnp.float32),
        pltpu.VMEM((K, D), jnp.float32),
        pltpu.VMEM((K, D), jnp.float32),
        pltpu.VMEM((K, D), jnp.float32),
        pltpu.VMEM((K, D), jnp.float32),
        pltpu.VMEM_SHARED((N, D), jnp.float32),
        pltpu.SemaphoreType.DMA,
        pltpu.SemaphoreType.DMA,
        pltpu.SemaphoreType.DMA,
        pltpu.SemaphoreType.DMA,
        pltpu.SemaphoreType.DMA,
        pltpu.SemaphoreType.DMA,
    ],
)(_sc_body)


def kernel(x, edge_index, edge_attr, W, b, bn_g, bn_b, n1_g, n1_b, W1, b1,
           W2, b2, n2_g, n2_b):
    src = edge_index[0].astype(jnp.int32).reshape(NW, EPT)
    dst = edge_index[1].astype(jnp.int32)

    h = pl.pallas_call(
        _lin_body,
        out_shape=jax.ShapeDtypeStruct((N, D), jnp.float32),
    )(x, W)

    partials = _sc_propagate(h, src, dst, edge_attr)

    out = pl.pallas_call(
        _post_body,
        out_shape=jax.ShapeDtypeStruct((N, D), jnp.float32),
    )(partials, x, b.reshape(1, D), bn_g.reshape(1, D), bn_b.reshape(1, D),
      n1_g.reshape(1, D), n1_b.reshape(1, D), W1, b1.reshape(1, 2 * D),
      W2, b2.reshape(1, D), n2_g.reshape(1, D), n2_b.reshape(1, D))
    return out
